# initial kernel scaffold (unmeasured)
import jax
import jax.numpy as jnp
from jax import lax
from jax.experimental import pallas as pl
from jax.experimental.pallas import tpu as pltpu


def kernel(
    x,
):
    def body(*refs):
        pass

    out_shape = jax.ShapeDtypeStruct(..., jnp.float32)
    return pl.pallas_call(body, out_shape=out_shape)(...)


# baseline (device time: 12691 ns/iter reference)
import jax
import jax.numpy as jnp
from jax import lax
from jax.experimental import pallas as pl
from jax.experimental.pallas import tpu as pltpu

N_DEV = 8


def kernel(x):
    m_per, n = x.shape
    inv_total = 1.0 / (N_DEV * m_per)

    def body(x_ref, out_ref, acc_ref, send_sems, recv_sems):
        my_pos = lax.axis_index("i")

        acc_ref[pl.ds(my_pos, 1), :] = jnp.sum(x_ref[:, :], axis=0, keepdims=True)

        for j in range(N_DEV):
            @pl.when(my_pos != j)
            def _send():
                rdma = pltpu.make_async_remote_copy(
                    src_ref=acc_ref.at[pl.ds(my_pos, 1), :],
                    dst_ref=acc_ref.at[pl.ds(my_pos, 1), :],
                    send_sem=send_sems.at[j],
                    recv_sem=recv_sems.at[my_pos],
                    device_id=(j,),
                    device_id_type=pl.DeviceIdType.MESH,
                )
                rdma.start()

        for j in range(N_DEV):
            @pl.when(my_pos != j)
            def _wait():
                rdma = pltpu.make_async_remote_copy(
                    src_ref=acc_ref.at[pl.ds(j, 1), :],
                    dst_ref=acc_ref.at[pl.ds(j, 1), :],
                    send_sem=send_sems.at[j],
                    recv_sem=recv_sems.at[j],
                    device_id=(j,),
                    device_id_type=pl.DeviceIdType.MESH,
                )
                rdma.wait_send()
                rdma.wait_recv()

        out_ref[:, :] = jnp.sum(acc_ref[:, :], axis=0, keepdims=True) * inv_total

    return pl.pallas_call(
        body,
        out_shape=jax.ShapeDtypeStruct((1, n), jnp.float32),
        in_specs=[pl.BlockSpec(memory_space=pltpu.VMEM)],
        out_specs=pl.BlockSpec(memory_space=pltpu.VMEM),
        scratch_shapes=[
            pltpu.VMEM((N_DEV, n), jnp.float32),
            pltpu.SemaphoreType.DMA((N_DEV,)),
            pltpu.SemaphoreType.DMA((N_DEV,)),
        ],
    )(x)


# device time: 8724 ns/iter; 1.4547x vs baseline; 1.4547x over previous
import jax
import jax.numpy as jnp
from jax import lax
from jax.experimental import pallas as pl
from jax.experimental.pallas import tpu as pltpu

N_DEV = 8


def kernel(x):
    m_per, n = x.shape
    inv_total = 1.0 / (N_DEV * m_per)

    def body(x_ref, out_ref, acc_ref, send_sems, recv_sems):
        my_pos = lax.axis_index("i")

        acc_ref[pl.ds(my_pos, 1), :] = jnp.sum(x_ref[:, :], axis=0, keepdims=True)

        barrier_sem = pltpu.get_barrier_semaphore()
        for j in range(N_DEV):
            @pl.when(my_pos != j)
            def _signal():
                pl.semaphore_signal(
                    barrier_sem, inc=1,
                    device_id=(j,), device_id_type=pl.DeviceIdType.MESH,
                )
        pl.semaphore_wait(barrier_sem, N_DEV - 1)

        for j in range(N_DEV):
            @pl.when(my_pos != j)
            def _send():
                rdma = pltpu.make_async_remote_copy(
                    src_ref=acc_ref.at[pl.ds(my_pos, 1), :],
                    dst_ref=acc_ref.at[pl.ds(my_pos, 1), :],
                    send_sem=send_sems.at[j],
                    recv_sem=recv_sems.at[my_pos],
                    device_id=(j,),
                    device_id_type=pl.DeviceIdType.MESH,
                )
                rdma.start()

        for j in range(N_DEV):
            @pl.when(my_pos != j)
            def _wait():
                rdma = pltpu.make_async_remote_copy(
                    src_ref=acc_ref.at[pl.ds(j, 1), :],
                    dst_ref=acc_ref.at[pl.ds(j, 1), :],
                    send_sem=send_sems.at[j],
                    recv_sem=recv_sems.at[j],
                    device_id=(j,),
                    device_id_type=pl.DeviceIdType.MESH,
                )
                rdma.wait_send()
                rdma.wait_recv()

        out_ref[:, :] = jnp.sum(acc_ref[:, :], axis=0, keepdims=True) * inv_total

    return pl.pallas_call(
        body,
        out_shape=jax.ShapeDtypeStruct((1, n), jnp.float32),
        in_specs=[pl.BlockSpec(memory_space=pltpu.VMEM)],
        out_specs=pl.BlockSpec(memory_space=pltpu.VMEM),
        scratch_shapes=[
            pltpu.VMEM((N_DEV, n), jnp.float32),
            pltpu.SemaphoreType.DMA((N_DEV,)),
            pltpu.SemaphoreType.DMA((N_DEV,)),
        ],
        compiler_params=pltpu.CompilerParams(collective_id=0),
    )(x)


# device time: 8530 ns/iter; 1.4878x vs baseline; 1.0227x over previous
import jax
import jax.numpy as jnp
from jax import lax
from jax.experimental import pallas as pl
from jax.experimental.pallas import tpu as pltpu

N_DEV = 8


def kernel(x):
    m_per, n = x.shape
    inv_total = 1.0 / (N_DEV * m_per)

    def body(x_ref, out_ref, acc_ref, send_sems, recv_sems):
        my_pos = lax.axis_index("i")

        barrier_sem = pltpu.get_barrier_semaphore()
        for j in range(N_DEV):
            @pl.when(my_pos != j)
            def _signal():
                pl.semaphore_signal(
                    barrier_sem, inc=1,
                    device_id=(j,), device_id_type=pl.DeviceIdType.MESH,
                )

        acc_ref[pl.ds(my_pos, 1), :] = jnp.sum(x_ref[:, :], axis=0, keepdims=True)

        pl.semaphore_wait(barrier_sem, N_DEV - 1)

        for j in range(N_DEV):
            @pl.when(my_pos != j)
            def _send():
                rdma = pltpu.make_async_remote_copy(
                    src_ref=acc_ref.at[pl.ds(my_pos, 1), :],
                    dst_ref=acc_ref.at[pl.ds(my_pos, 1), :],
                    send_sem=send_sems.at[j],
                    recv_sem=recv_sems.at[my_pos],
                    device_id=(j,),
                    device_id_type=pl.DeviceIdType.MESH,
                )
                rdma.start()

        for j in range(N_DEV):
            @pl.when(my_pos != j)
            def _wait():
                rdma = pltpu.make_async_remote_copy(
                    src_ref=acc_ref.at[pl.ds(j, 1), :],
                    dst_ref=acc_ref.at[pl.ds(j, 1), :],
                    send_sem=send_sems.at[j],
                    recv_sem=recv_sems.at[j],
                    device_id=(j,),
                    device_id_type=pl.DeviceIdType.MESH,
                )
                rdma.wait_send()
                rdma.wait_recv()

        out_ref[:, :] = jnp.sum(acc_ref[:, :], axis=0, keepdims=True) * inv_total

    return pl.pallas_call(
        body,
        out_shape=jax.ShapeDtypeStruct((1, n), jnp.float32),
        in_specs=[pl.BlockSpec(memory_space=pltpu.VMEM)],
        out_specs=pl.BlockSpec(memory_space=pltpu.VMEM),
        scratch_shapes=[
            pltpu.VMEM((N_DEV, n), jnp.float32),
            pltpu.SemaphoreType.DMA((N_DEV,)),
            pltpu.SemaphoreType.DMA((N_DEV,)),
        ],
        compiler_params=pltpu.CompilerParams(collective_id=0),
    )(x)
